# baseline (speedup 1.0000x reference)
import functools

import jax
import jax.numpy as jnp
from jax import lax
from jax.experimental import pallas as pl
from jax.experimental.pallas import tpu as pltpu


def _pick_block(n, target):
    best = None
    for b in range(8, target + 1, 8):
        if n % b == 0:
            best = b
    if best is None:
        raise ValueError(f"no block for n={n}")
    return best


def _mm_body(x_ref, w_ref, o_ref):
    r = jnp.dot(x_ref[...], w_ref[...], preferred_element_type=jnp.float32)
    o_ref[...] = r.astype(o_ref.dtype)


def _matmul(x, w, block, out_dtype=jnp.float32):
    n, k = x.shape
    f = w.shape[1]
    return pl.pallas_call(
        _mm_body,
        grid=(n // block,),
        in_specs=[
            pl.BlockSpec((block, k), lambda i: (i, 0)),
            pl.BlockSpec((k, f), lambda i: (0, 0)),
        ],
        out_specs=pl.BlockSpec((block, f), lambda i: (i, 0)),
        out_shape=jax.ShapeDtypeStruct((n, f), out_dtype),
    )(x, w)


def _agg_body_first(adj_ref, sup_ref, w_ref, abf_ref, o_ref):
    a = adj_ref[...].astype(jnp.bfloat16)
    abf_ref[...] = a
    z = jnp.dot(a, sup_ref[...], preferred_element_type=jnp.float32)
    z = jnp.maximum(z, 0.0)
    r = jnp.dot(z, w_ref[...], preferred_element_type=jnp.float32)
    o_ref[...] = r.astype(o_ref.dtype)


def _agg_first(adj, sup, w_next, block):
    n = adj.shape[0]
    f = sup.shape[1]
    fn = w_next.shape[1]
    adj_bf, sup_next = pl.pallas_call(
        _agg_body_first,
        grid=(n // block,),
        in_specs=[
            pl.BlockSpec((block, n), lambda i: (i, 0)),
            pl.BlockSpec((n, f), lambda i: (0, 0)),
            pl.BlockSpec((f, fn), lambda i: (0, 0)),
        ],
        out_specs=[
            pl.BlockSpec((block, n), lambda i: (i, 0)),
            pl.BlockSpec((block, fn), lambda i: (i, 0)),
        ],
        out_shape=[
            jax.ShapeDtypeStruct((n, n), jnp.bfloat16),
            jax.ShapeDtypeStruct((n, fn), jnp.bfloat16),
        ],
    )(adj, sup, w_next)
    return adj_bf, sup_next


def _agg_body_sup(adj_ref, sup_ref, w_ref, o_ref, *, relu):
    z = jnp.dot(adj_ref[...], sup_ref[...], preferred_element_type=jnp.float32)
    if relu:
        z = jnp.maximum(z, 0.0)
    r = jnp.dot(z, w_ref[...], preferred_element_type=jnp.float32)
    o_ref[...] = r.astype(o_ref.dtype)


def _agg_body_z(adj_ref, sup_ref, z_ref, *, relu):
    z = jnp.dot(adj_ref[...], sup_ref[...], preferred_element_type=jnp.float32)
    if relu:
        z = jnp.maximum(z, 0.0)
    z_ref[...] = z.astype(z_ref.dtype)


def _agg_body_both(adj_ref, sup_ref, w_ref, z_ref, o_ref, *, relu):
    z = jnp.dot(adj_ref[...], sup_ref[...], preferred_element_type=jnp.float32)
    if relu:
        z = jnp.maximum(z, 0.0)
    z_ref[...] = z.astype(z_ref.dtype)
    r = jnp.dot(z, w_ref[...], preferred_element_type=jnp.float32)
    o_ref[...] = r.astype(o_ref.dtype)


def _agg_pass(adj, sup, w_next, relu, want_z, block, sup_dtype=jnp.float32):
    n = adj.shape[0]
    f = sup.shape[1]
    in_specs = [
        pl.BlockSpec((block, n), lambda i: (i, 0)),
        pl.BlockSpec((n, f), lambda i: (0, 0)),
    ]
    args = [adj, sup]
    out_specs = []
    out_shape = []
    if want_z:
        out_specs.append(pl.BlockSpec((block, f), lambda i: (i, 0)))
        out_shape.append(jax.ShapeDtypeStruct((n, f), jnp.float32))
    if w_next is not None:
        fn = w_next.shape[1]
        in_specs.append(pl.BlockSpec((f, fn), lambda i: (0, 0)))
        args.append(w_next)
        out_specs.append(pl.BlockSpec((block, fn), lambda i: (i, 0)))
        out_shape.append(jax.ShapeDtypeStruct((n, fn), sup_dtype))
    if want_z and w_next is not None:
        body = functools.partial(_agg_body_both, relu=relu)
    elif want_z:
        body = functools.partial(_agg_body_z, relu=relu)
    else:
        body = functools.partial(_agg_body_sup, relu=relu)
    out = pl.pallas_call(
        body,
        grid=(n // block,),
        in_specs=in_specs,
        out_specs=out_specs,
        out_shape=out_shape,
    )(*args)
    return out[0] if len(out) == 1 else out


def _mega(adj_bf, sup2, W3, W4, W5, W6, W7, W8, block):
    n = adj_bf.shape[0]
    nb = n // block
    Ws = (W3, W4, W5, W6, W7, W8)
    fg = W5.shape[0]
    fh = W8.shape[1]
    maxf = max(w.shape[1] for w in Ws)

    def body(adj_ref, sup2_ref, w3_ref, w4_ref, w5_ref, w6_ref, w7_ref,
             w8_ref, zg_ref, zh_ref, p0, p1):
        l = pl.program_id(0)
        i = pl.program_id(1)
        a = adj_ref[...]
        row = pl.ds(i * block, block)
        wrefs = (w3_ref, w4_ref, w5_ref, w6_ref, w7_ref, w8_ref)
        srcs = (None, p1, p0, p1, p0, p1, p0)
        dsts = (p1, p0, p1, p0, p1, p0, None)

        def mk(k):
            def br():
                if k == 0:
                    sup = sup2_ref[...]
                else:
                    f_in = wrefs[k - 1].shape[1] if k <= 6 else None
                    sup = srcs[k][:, :f_in]
                z = jnp.dot(a, sup, preferred_element_type=jnp.float32)
                if k != 2:
                    z = jnp.maximum(z, 0.0)
                if k == 2:
                    zg_ref[...] = z
                if k == 6:
                    zh_ref[...] = z
                if k < 6:
                    w = wrefs[k]
                    s = jnp.dot(z, w[...], preferred_element_type=jnp.float32)
                    dsts[k][row, :w.shape[1]] = s.astype(jnp.bfloat16)
            return br

        lax.switch(l, [mk(k) for k in range(7)])

    zg, zh = pl.pallas_call(
        body,
        grid=(7, nb),
        in_specs=[
            pl.BlockSpec((block, n), lambda l, i: (i, 0)),
            pl.BlockSpec(sup2.shape, lambda l, i: (0, 0)),
        ] + [pl.BlockSpec(w.shape, lambda l, i: (0, 0)) for w in Ws],
        out_specs=[
            pl.BlockSpec((block, fg), lambda l, i: (jnp.where(l == 2, i, nb), 0)),
            pl.BlockSpec((block, fh), lambda l, i: (jnp.where(l == 6, i, nb), 0)),
        ],
        out_shape=[
            jax.ShapeDtypeStruct((n + block, fg), jnp.float32),
            jax.ShapeDtypeStruct((n + block, fh), jnp.float32),
        ],
        scratch_shapes=[
            pltpu.VMEM((n, maxf), jnp.bfloat16),
            pltpu.VMEM((n, maxf), jnp.bfloat16),
        ],
    )(adj_bf, sup2, *Ws)
    return zg[:n], zh[:n]


def _sigmoid(x):
    return 1.0 / (1.0 + jnp.exp(-x))


def _adjhat_body(zgi_ref, zhi_ref, zgt_ref, zht_ref, o_ref):
    zgi = zgi_ref[...].astype(jnp.bfloat16)
    zhi = zhi_ref[...].astype(jnp.bfloat16)
    a = jnp.dot(zgi, zgt_ref[...], preferred_element_type=jnp.float32)
    b = jnp.dot(zhi, zht_ref[...], preferred_element_type=jnp.float32)
    o_ref[...] = 1.0 + 0.5 * (jnp.tanh(0.5 * a) + jnp.tanh(0.5 * b))


def _adjhat(z_gae, z_hat, block):
    n, fg = z_gae.shape
    fh = z_hat.shape[1]
    zgt = z_gae.T.astype(jnp.bfloat16)
    zht = z_hat.T.astype(jnp.bfloat16)
    return pl.pallas_call(
        _adjhat_body,
        grid=(n // block,),
        in_specs=[
            pl.BlockSpec((block, fg), lambda i: (i, 0)),
            pl.BlockSpec((block, fh), lambda i: (i, 0)),
            pl.BlockSpec((fg, n), lambda i: (0, 0)),
            pl.BlockSpec((fh, n), lambda i: (0, 0)),
        ],
        out_specs=pl.BlockSpec((block, n), lambda i: (i, 0)),
        out_shape=jax.ShapeDtypeStruct((n, n), jnp.float32),
    )(z_gae, z_hat, zgt, zht)


def kernel(x, adj, W1, W2, W3, W4, W5, W6, W7, W8):
    n = adj.shape[0]
    bf = jnp.bfloat16
    sup1 = _matmul(x, W1, _pick_block(n, 1000), out_dtype=bf)
    adj_bf, sup2 = _agg_first(adj, sup1, W2, _pick_block(n, 400))
    z_gae, z_hat = _mega(adj_bf, sup2, W3, W4, W5, W6, W7, W8,
                         _pick_block(n, 400))
    adj_hat = _adjhat(z_gae, z_hat, _pick_block(n, 400))
    return (z_gae, z_hat, adj_hat)


# speedup vs baseline: 1.1313x; 1.1313x over previous
import functools

import jax
import jax.numpy as jnp
from jax import lax
from jax.experimental import pallas as pl
from jax.experimental.pallas import tpu as pltpu


def _pick_block(n, target):
    best = None
    for b in range(8, target + 1, 8):
        if n % b == 0:
            best = b
    if best is None:
        raise ValueError(f"no block for n={n}")
    return best


def _mm_body(x_ref, w_ref, o_ref):
    r = jnp.dot(x_ref[...], w_ref[...], preferred_element_type=jnp.float32)
    o_ref[...] = r.astype(o_ref.dtype)


def _matmul(x, w, block, out_dtype=jnp.float32):
    n, k = x.shape
    f = w.shape[1]
    return pl.pallas_call(
        _mm_body,
        grid=(n // block,),
        in_specs=[
            pl.BlockSpec((block, k), lambda i: (i, 0)),
            pl.BlockSpec((k, f), lambda i: (0, 0)),
        ],
        out_specs=pl.BlockSpec((block, f), lambda i: (i, 0)),
        out_shape=jax.ShapeDtypeStruct((n, f), out_dtype),
    )(x, w)


def _agg_body_first(adj_ref, sup_ref, w_ref, abf_ref, o_ref):
    a = adj_ref[...].astype(jnp.bfloat16)
    abf_ref[...] = a
    z = jnp.dot(a, sup_ref[...], preferred_element_type=jnp.float32)
    z = jnp.maximum(z, 0.0)
    r = jnp.dot(z, w_ref[...], preferred_element_type=jnp.float32)
    o_ref[...] = r.astype(o_ref.dtype)


def _agg_first(adj, sup, w_next, block):
    n = adj.shape[0]
    f = sup.shape[1]
    fn = w_next.shape[1]
    adj_bf, sup_next = pl.pallas_call(
        _agg_body_first,
        grid=(n // block,),
        in_specs=[
            pl.BlockSpec((block, n), lambda i: (i, 0)),
            pl.BlockSpec((n, f), lambda i: (0, 0)),
            pl.BlockSpec((f, fn), lambda i: (0, 0)),
        ],
        out_specs=[
            pl.BlockSpec((block, n), lambda i: (i, 0)),
            pl.BlockSpec((block, fn), lambda i: (i, 0)),
        ],
        out_shape=[
            jax.ShapeDtypeStruct((n, n), jnp.bfloat16),
            jax.ShapeDtypeStruct((n, fn), jnp.bfloat16),
        ],
    )(adj, sup, w_next)
    return adj_bf, sup_next


def _agg_body_sup(adj_ref, sup_ref, w_ref, o_ref, *, relu):
    z = jnp.dot(adj_ref[...], sup_ref[...], preferred_element_type=jnp.float32)
    if relu:
        z = jnp.maximum(z, 0.0)
    r = jnp.dot(z, w_ref[...], preferred_element_type=jnp.float32)
    o_ref[...] = r.astype(o_ref.dtype)


def _agg_body_z(adj_ref, sup_ref, z_ref, *, relu):
    z = jnp.dot(adj_ref[...], sup_ref[...], preferred_element_type=jnp.float32)
    if relu:
        z = jnp.maximum(z, 0.0)
    z_ref[...] = z.astype(z_ref.dtype)


def _agg_body_both(adj_ref, sup_ref, w_ref, z_ref, o_ref, *, relu):
    z = jnp.dot(adj_ref[...], sup_ref[...], preferred_element_type=jnp.float32)
    if relu:
        z = jnp.maximum(z, 0.0)
    z_ref[...] = z.astype(z_ref.dtype)
    r = jnp.dot(z, w_ref[...], preferred_element_type=jnp.float32)
    o_ref[...] = r.astype(o_ref.dtype)


def _agg_pass(adj, sup, w_next, relu, want_z, block, sup_dtype=jnp.float32):
    n = adj.shape[0]
    f = sup.shape[1]
    in_specs = [
        pl.BlockSpec((block, n), lambda i: (i, 0)),
        pl.BlockSpec((n, f), lambda i: (0, 0)),
    ]
    args = [adj, sup]
    out_specs = []
    out_shape = []
    if want_z:
        out_specs.append(pl.BlockSpec((block, f), lambda i: (i, 0)))
        out_shape.append(jax.ShapeDtypeStruct((n, f), jnp.float32))
    if w_next is not None:
        fn = w_next.shape[1]
        in_specs.append(pl.BlockSpec((f, fn), lambda i: (0, 0)))
        args.append(w_next)
        out_specs.append(pl.BlockSpec((block, fn), lambda i: (i, 0)))
        out_shape.append(jax.ShapeDtypeStruct((n, fn), sup_dtype))
    if want_z and w_next is not None:
        body = functools.partial(_agg_body_both, relu=relu)
    elif want_z:
        body = functools.partial(_agg_body_z, relu=relu)
    else:
        body = functools.partial(_agg_body_sup, relu=relu)
    out = pl.pallas_call(
        body,
        grid=(n // block,),
        in_specs=in_specs,
        out_specs=out_specs,
        out_shape=out_shape,
    )(*args)
    return out[0] if len(out) == 1 else out


def _mega(adj_bf, sup2, W3, W4, W5, W6, W7, W8, block):
    n = adj_bf.shape[0]
    nb = n // block
    Ws = (W3, W4, W5, W6, W7, W8)
    fg = W5.shape[0]
    fh = W8.shape[1]
    maxf = max(w.shape[1] for w in Ws)

    def body(adj_ref, sup2_ref, w3_ref, w4_ref, w5_ref, w6_ref, w7_ref,
             w8_ref, zg_ref, zh_ref, p0, p1):
        l = pl.program_id(0)
        i = pl.program_id(1)
        a = adj_ref[...]
        row = pl.ds(i * block, block)
        wrefs = (w3_ref, w4_ref, w5_ref, w6_ref, w7_ref, w8_ref)
        srcs = (None, p1, p0, p1, p0, p1, p0)
        dsts = (p1, p0, p1, p0, p1, p0, None)

        def mk(k):
            def br():
                if k == 0:
                    sup = sup2_ref[...]
                else:
                    f_in = wrefs[k - 1].shape[1] if k <= 6 else None
                    sup = srcs[k][:, :f_in]
                z = jnp.dot(a, sup, preferred_element_type=jnp.float32)
                if k != 2:
                    z = jnp.maximum(z, 0.0)
                if k == 2:
                    zg_ref[...] = z
                if k == 6:
                    zh_ref[...] = z
                if k < 6:
                    w = wrefs[k]
                    s = jnp.dot(z, w[...], preferred_element_type=jnp.float32)
                    dsts[k][row, :w.shape[1]] = s.astype(jnp.bfloat16)
            return br

        lax.switch(l, [mk(k) for k in range(7)])

    zg, zh = pl.pallas_call(
        body,
        grid=(7, nb),
        in_specs=[
            pl.BlockSpec((block, n), lambda l, i: (i, 0)),
            pl.BlockSpec(sup2.shape, lambda l, i: (0, 0)),
        ] + [pl.BlockSpec(w.shape, lambda l, i: (0, 0)) for w in Ws],
        out_specs=[
            pl.BlockSpec((block, fg), lambda l, i: (jnp.where(l == 2, i, nb), 0)),
            pl.BlockSpec((block, fh), lambda l, i: (jnp.where(l == 6, i, nb), 0)),
        ],
        out_shape=[
            jax.ShapeDtypeStruct((n + block, fg), jnp.float32),
            jax.ShapeDtypeStruct((n + block, fh), jnp.float32),
        ],
        scratch_shapes=[
            pltpu.VMEM((n, maxf), jnp.bfloat16),
            pltpu.VMEM((n, maxf), jnp.bfloat16),
        ],
    )(adj_bf, sup2, *Ws)
    return zg[:n], zh[:n]


def _agg_pair(adj_bf, sup_in, Wmid, Wnext, relu0, relu1, want_z0, block):
    n = adj_bf.shape[0]
    nb = n // block
    f0 = sup_in.shape[1]
    fmid = Wmid.shape[1]
    f2 = Wnext.shape[1]

    def body(adj_ref, sup_ref, wm_ref, wn_ref, *rest):
        if want_z0:
            zg_ref, out_ref, scr = rest
        else:
            out_ref, scr = rest
        s = pl.program_id(0)
        i = pl.program_id(1)
        a = adj_ref[...]
        row = pl.ds(i * block, block)

        def sub0():
            z = jnp.dot(a, sup_ref[...], preferred_element_type=jnp.float32)
            zz = jnp.maximum(z, 0.0) if relu0 else z
            if want_z0:
                zg_ref[...] = zz
            m = jnp.dot(zz, wm_ref[...], preferred_element_type=jnp.float32)
            scr[row, :fmid] = m.astype(jnp.bfloat16)

        def sub1():
            z = jnp.dot(a, scr[:, :fmid], preferred_element_type=jnp.float32)
            if relu1:
                z = jnp.maximum(z, 0.0)
            o = jnp.dot(z, wn_ref[...], preferred_element_type=jnp.float32)
            out_ref[...] = o.astype(jnp.bfloat16)

        lax.cond(s == 0, sub0, sub1)

    out_specs = []
    out_shape = []
    if want_z0:
        out_specs.append(
            pl.BlockSpec((block, f0), lambda s, i: (jnp.where(s == 0, i, nb), 0)))
        out_shape.append(jax.ShapeDtypeStruct((n + block, f0), jnp.float32))
    out_specs.append(
        pl.BlockSpec((block, f2), lambda s, i: (jnp.where(s == 1, i, nb), 0)))
    out_shape.append(jax.ShapeDtypeStruct((n + block, f2), jnp.bfloat16))

    out = pl.pallas_call(
        body,
        grid=(2, nb),
        in_specs=[
            pl.BlockSpec((block, n), lambda s, i: (i, 0)),
            pl.BlockSpec((n, f0), lambda s, i: (0, 0)),
            pl.BlockSpec(Wmid.shape, lambda s, i: (0, 0)),
            pl.BlockSpec(Wnext.shape, lambda s, i: (0, 0)),
        ],
        out_specs=out_specs,
        out_shape=out_shape,
        scratch_shapes=[pltpu.VMEM((n, fmid), jnp.bfloat16)],
    )(adj_bf, sup_in, Wmid, Wnext)
    if want_z0:
        return out[0][:n], out[1][:n]
    return out[0][:n]


def _sigmoid(x):
    return 1.0 / (1.0 + jnp.exp(-x))


_NT = (((1,), (1,)), ((), ()))


def _adjhat_body(zgi_ref, zhi_ref, zg_ref, zh_ref, o_ref):
    zgi = zgi_ref[...].astype(jnp.bfloat16)
    zg = zg_ref[...].astype(jnp.bfloat16)
    a = lax.dot_general(zgi, zg, _NT, preferred_element_type=jnp.float32)
    b = lax.dot_general(zhi_ref[...], zh_ref[...], _NT,
                        preferred_element_type=jnp.float32)
    o_ref[...] = 1.0 + 0.5 * (jnp.tanh(0.5 * a) + jnp.tanh(0.5 * b))


def _adjhat(z_gae, z_hat, block):
    n, fg = z_gae.shape
    fh = z_hat.shape[1]
    zh_bf = z_hat.astype(jnp.bfloat16)
    return pl.pallas_call(
        _adjhat_body,
        grid=(n // block,),
        in_specs=[
            pl.BlockSpec((block, fg), lambda i: (i, 0)),
            pl.BlockSpec((block, fh), lambda i: (i, 0)),
            pl.BlockSpec((n, fg), lambda i: (0, 0)),
            pl.BlockSpec((n, fh), lambda i: (0, 0)),
        ],
        out_specs=pl.BlockSpec((block, n), lambda i: (i, 0)),
        out_shape=jax.ShapeDtypeStruct((n, n), jnp.float32),
    )(z_gae, zh_bf, z_gae, zh_bf)


def kernel(x, adj, W1, W2, W3, W4, W5, W6, W7, W8):
    n = adj.shape[0]
    bf = jnp.bfloat16
    blk = _pick_block(n, 1000)
    sup1 = _matmul(x, W1, _pick_block(n, 1000), out_dtype=bf)
    adj_bf, sup2 = _agg_first(adj, sup1, W2, _pick_block(n, 400))
    sup3 = _agg_pass(adj_bf, sup2, W3, relu=True, want_z=False, block=blk, sup_dtype=bf)
    sup4 = _agg_pass(adj_bf, sup3, W4, relu=True, want_z=False, block=blk, sup_dtype=bf)
    z_gae, sup5 = _agg_pass(adj_bf, sup4, W5, relu=False, want_z=True, block=blk, sup_dtype=bf)
    sup6 = _agg_pass(adj_bf, sup5, W6, relu=True, want_z=False, block=blk, sup_dtype=bf)
    sup7 = _agg_pass(adj_bf, sup6, W7, relu=True, want_z=False, block=blk, sup_dtype=bf)
    sup8 = _agg_pass(adj_bf, sup7, W8, relu=True, want_z=False, block=blk, sup_dtype=bf)
    z_hat = _agg_pass(adj_bf, sup8, None, relu=True, want_z=True, block=blk)
    adj_hat = _adjhat(z_gae, z_hat, _pick_block(n, 400))
    return (z_gae, z_hat, adj_hat)
